# baseline (speedup 1.0000x reference)
import functools

import jax
import jax.numpy as jnp
from jax import lax
from jax.experimental import pallas as pl
from jax.experimental.pallas import tpu as pltpu
from jax.experimental.pallas import tpu_sc as plsc

_NC, _NS, _L = 2, 16, 16
_NW = _NC * _NS
_P = 32


def kernel(x, pos_table):
    batch, seq, d = x.shape
    pos_per_w = seq // _NW
    nchunks = pos_per_w // _P
    cw = _P * d

    x2 = x.reshape(batch, seq * d)
    pos1 = pos_table.reshape(-1)

    mesh = plsc.VectorSubcoreMesh(core_axis_name="c", subcore_axis_name="s")

    @functools.partial(
        pl.kernel,
        out_type=jax.ShapeDtypeStruct((batch, seq * d), x.dtype),
        mesh=mesh,
        scratch_types=[
            pltpu.VMEM((cw,), jnp.float32),
            pltpu.VMEM((cw,), jnp.float32),
        ],
    )
    def sc_add(x_hbm, pos_hbm, out_hbm, pos_v, x_v):
        wid = lax.axis_index("s") * _NC + lax.axis_index("c")
        base = wid * (pos_per_w * d)
        for c in range(nchunks):
            off = base + c * cw
            pltpu.sync_copy(pos_hbm.at[pl.ds(off, cw)], pos_v)
            for b in range(batch):
                pltpu.sync_copy(x_hbm.at[b, pl.ds(off, cw)], x_v)

                @plsc.parallel_loop(0, cw, step=_L, unroll=8)
                def _body(i):
                    x_v[pl.ds(i, _L)] = x_v[pl.ds(i, _L)] + pos_v[pl.ds(i, _L)]

                pltpu.sync_copy(x_v, out_hbm.at[b, pl.ds(off, cw)])

    out = sc_add(x2, pos1)
    return out.reshape(batch, seq, d)


# speedup vs baseline: 1.1783x; 1.1783x over previous
import functools

import jax
import jax.numpy as jnp
from jax import lax
from jax.experimental import pallas as pl
from jax.experimental.pallas import tpu as pltpu
from jax.experimental.pallas import tpu_sc as plsc

_NC, _NS, _L = 2, 16, 16
_NW = _NC * _NS
_P = 16
_NSLOT = 4


def kernel(x, pos_table):
    batch, seq, d = x.shape
    pos_per_w = seq // _NW
    nchunks = pos_per_w // _P
    cw = _P * d
    steps = nchunks * batch

    x2 = x.reshape(batch, seq * d)
    pos1 = pos_table.reshape(-1)

    mesh = plsc.VectorSubcoreMesh(core_axis_name="c", subcore_axis_name="s")

    @functools.partial(
        pl.kernel,
        out_type=jax.ShapeDtypeStruct((batch, seq * d), x.dtype),
        mesh=mesh,
        scratch_types=(
            [pltpu.VMEM((cw,), jnp.float32) for _ in range(_NSLOT)]
            + [pltpu.VMEM((cw,), jnp.float32) for _ in range(2)]
            + [pltpu.SemaphoreType.DMA for _ in range(_NSLOT * 2 + 2)]
        ),
    )
    def sc_add(x_hbm, pos_hbm, out_hbm, *scr):
        xb = scr[0:_NSLOT]
        pb = scr[_NSLOT:_NSLOT + 2]
        isem = scr[_NSLOT + 2:_NSLOT * 2 + 2]
        osem = scr[_NSLOT * 2 + 2:_NSLOT * 3 + 2]
        psem = scr[_NSLOT * 3 + 2:]

        wid = lax.axis_index("s") * _NC + lax.axis_index("c")
        base = wid * (pos_per_w * d)

        def in_copy(t):
            c, b = divmod(t, batch)
            s = t % _NSLOT
            return pltpu.async_copy(
                x_hbm.at[b, pl.ds(base + c * cw, cw)], xb[s], isem[s])

        def out_copy(t):
            c, b = divmod(t, batch)
            s = t % _NSLOT
            return pltpu.async_copy(
                xb[s], out_hbm.at[b, pl.ds(base + c * cw, cw)], osem[s])

        def pos_copy(c):
            return pltpu.async_copy(
                pos_hbm.at[pl.ds(base + c * cw, cw)], pb[c % 2], psem[c % 2])

        pos_h, in_h, out_h = {}, {}, {}
        waited_out = set()

        for c in range(min(2, nchunks)):
            pos_h[c] = pos_copy(c)
        for t in range(min(_NSLOT - 1, steps)):
            in_h[t] = in_copy(t)

        for t in range(steps):
            s = t % _NSLOT
            c, b = divmod(t, batch)
            in_h[t].wait()
            if b == 0:
                pos_h[c].wait()
            xv, pv = xb[s], pb[c % 2]

            @plsc.parallel_loop(0, cw, step=_L, unroll=8)
            def _body(i, xv=xv, pv=pv):
                plsc.addupdate(xv.at[pl.ds(i, _L)], pv[pl.ds(i, _L)])

            out_h[t] = out_copy(t)
            out_h[t].wait()
            waited_out.add(t)
            if b == batch - 1 and c + 2 < nchunks:
                pos_h[c + 2] = pos_copy(c + 2)
            nt = t + _NSLOT - 1
            if nt < steps:
                in_h[nt] = in_copy(nt)

        for t in range(steps):
            if t not in waited_out:
                out_h[t].wait()

    out = sc_add(x2, pos1)
    return out.reshape(batch, seq, d)


# speedup vs baseline: 1.2260x; 1.0405x over previous
import functools

import jax
import jax.numpy as jnp
from jax import lax
from jax.experimental import pallas as pl
from jax.experimental.pallas import tpu as pltpu
from jax.experimental.pallas import tpu_sc as plsc

_NC, _NS, _L = 2, 16, 16
_NW = _NC * _NS
_P = 16
_NSLOT = 5
_DEPTH = 3


def kernel(x, pos_table):
    batch, seq, d = x.shape
    pos_per_w = seq // _NW
    nchunks = pos_per_w // _P
    cw = _P * d
    steps = nchunks * batch

    x2 = x.reshape(batch, seq * d)
    pos1 = pos_table.reshape(-1)

    mesh = plsc.VectorSubcoreMesh(core_axis_name="c", subcore_axis_name="s")

    @functools.partial(
        pl.kernel,
        out_type=jax.ShapeDtypeStruct((batch, seq * d), x.dtype),
        mesh=mesh,
        scratch_types=(
            [pltpu.VMEM((cw,), jnp.float32) for _ in range(_NSLOT)]
            + [pltpu.VMEM((cw,), jnp.float32) for _ in range(2)]
            + [pltpu.SemaphoreType.DMA for _ in range(_NSLOT * 2 + 2)]
        ),
    )
    def sc_add(x_hbm, pos_hbm, out_hbm, *scr):
        xb = scr[0:_NSLOT]
        pb = scr[_NSLOT:_NSLOT + 2]
        isem = scr[_NSLOT + 2:_NSLOT * 2 + 2]
        osem = scr[_NSLOT * 2 + 2:_NSLOT * 3 + 2]
        psem = scr[_NSLOT * 3 + 2:]

        wid = lax.axis_index("s") * _NC + lax.axis_index("c")
        base = wid * (pos_per_w * d)

        def in_copy(t):
            c, b = divmod(t, batch)
            s = t % _NSLOT
            return pltpu.async_copy(
                x_hbm.at[b, pl.ds(base + c * cw, cw)], xb[s], isem[s])

        def out_copy(t):
            c, b = divmod(t, batch)
            s = t % _NSLOT
            return pltpu.async_copy(
                xb[s], out_hbm.at[b, pl.ds(base + c * cw, cw)], osem[s])

        def pos_copy(c):
            return pltpu.async_copy(
                pos_hbm.at[pl.ds(base + c * cw, cw)], pb[c % 2], psem[c % 2])

        pos_h, in_h, out_h = {}, {}, {}
        waited_out = set()

        for c in range(min(2, nchunks)):
            pos_h[c] = pos_copy(c)
        for t in range(min(_DEPTH, steps)):
            in_h[t] = in_copy(t)

        for t in range(steps):
            s = t % _NSLOT
            c, b = divmod(t, batch)
            in_h[t].wait()
            if b == 0:
                pos_h[c].wait()
            xv, pv = xb[s], pb[c % 2]

            @plsc.parallel_loop(0, cw, step=_L, unroll=8)
            def _body(i, xv=xv, pv=pv):
                plsc.addupdate(xv.at[pl.ds(i, _L)], pv[pl.ds(i, _L)])

            if t >= 1:
                out_h[t - 1] = out_copy(t - 1)
            if b == batch - 1 and c + 2 < nchunks:
                pos_h[c + 2] = pos_copy(c + 2)
            nt = t + _DEPTH
            if nt < steps:
                pt = nt - _NSLOT
                if pt >= 0:
                    out_h[pt].wait()
                    waited_out.add(pt)
                in_h[nt] = in_copy(nt)

        out_h[steps - 1] = out_copy(steps - 1)
        for t in range(steps):
            if t not in waited_out:
                out_h[t].wait()

    out = sc_add(x2, pos1)
    return out.reshape(batch, seq, d)
